# baseline (speedup 1.0000x reference)
import functools

import jax
import jax.numpy as jnp
from jax import lax
from jax.experimental import pallas as pl
from jax.experimental.pallas import tpu as pltpu
from jax.experimental.pallas import tpu_sc as plsc

_N = 10000
_D = 128
_HID = 32
_G = 128
_NCORES = 2
_NSUB = 16
_NTILES = _NCORES * _NSUB
_CHUNK = 128
_K = 81
_EPAD = _NTILES * _K * _CHUNK
_NP = 10112
_WB = _NP // _NSUB
_YSTAGE = _N - (_NSUB - 1) * _WB


def _zero_acc_slice(zbuf, acc, base):
    for k in range(_WB // _CHUNK):
        pltpu.sync_copy(zbuf, acc.at[pl.ds(base + k * _CHUNK, _CHUNK)])
    rem = _WB - (_WB // _CHUNK) * _CHUNK
    pltpu.sync_copy(zbuf.at[pl.ds(0, rem)],
                    acc.at[pl.ds(base + _WB - rem, rem)])


def _zero_vmem(buf, width):
    zv = jnp.zeros((16,), jnp.float32)

    def zrow(i, carry):
        for off in range(0, width, 16):
            buf[i, pl.ds(off, 16)] = zv
        return carry

    lax.fori_loop(0, _CHUNK, zrow, 0)


_HD = _D // 2
_E = 320000


def _sc_scatter_seq(x, src_r, cont_r, scat_r, width):
    nv = width // 16
    stage = width == _HID
    mesh = plsc.VectorSubcoreMesh(core_axis_name="c", subcore_axis_name="s")

    scratch = [
        pltpu.VMEM((_K, _CHUNK), jnp.int32),
        pltpu.VMEM((_K, _CHUNK), jnp.float32),
        pltpu.VMEM((_K, _CHUNK), jnp.int32),
        pltpu.VMEM((_CHUNK, width), jnp.float32),
        pltpu.VMEM((_CHUNK, width), jnp.float32),
    ]
    if stage:
        scratch += [pltpu.VMEM((_WB, width), jnp.float32),
                    pltpu.VMEM_SHARED((_N, width), jnp.float32)]
    scratch += [pltpu.VMEM_SHARED((_NP, width), jnp.float32),
                pltpu.SemaphoreType.DMA]

    @functools.partial(
        pl.kernel,
        mesh=mesh,
        compiler_params=pltpu.CompilerParams(use_tc_tiling_on_sc=False,
                                             needs_layout_passes=False),
        out_type=jax.ShapeDtypeStruct((_NCORES, _NP, width), jnp.float32),
        scratch_types=scratch,
    )
    def scat(x_hbm, src_hbm, cont_hbm, scat_hbm, out_hbm, *refs):
        if stage:
            src_v, cont_v, scat_v, rows_v, zbuf, stage_v, x_sh, acc, sem = refs
        else:
            src_v, cont_v, scat_v, rows_v, zbuf, acc, sem = refs
            x_sh = x_hbm
        c = lax.axis_index("c")
        s = lax.axis_index("s")
        wid = c * _NSUB + s
        base = s * _WB

        _zero_vmem(zbuf, width)
        _zero_acc_slice(zbuf, acc, base)

        if stage:
            @pl.when(s < _NSUB - 1)
            def _():
                pltpu.sync_copy(x_hbm.at[pl.ds(base, _WB)], stage_v)
                pltpu.sync_copy(stage_v, x_sh.at[pl.ds(base, _WB)])

            @pl.when(s == _NSUB - 1)
            def _():
                pltpu.sync_copy(x_hbm.at[pl.ds((_NSUB - 1) * _WB, _YSTAGE)],
                                stage_v.at[pl.ds(0, _YSTAGE)])
                pltpu.sync_copy(stage_v.at[pl.ds(0, _YSTAGE)],
                                x_sh.at[pl.ds((_NSUB - 1) * _WB, _YSTAGE)])

        pltpu.sync_copy(src_hbm.at[wid], src_v)
        pltpu.sync_copy(cont_hbm.at[wid], cont_v)
        pltpu.sync_copy(scat_hbm.at[wid], scat_v)
        plsc.subcore_barrier()

        def window(j, accs):
            pltpu.async_copy(x_sh.at[src_v.at[j]], rows_v, sem).wait()
            jdx = jnp.full((16,), j, jnp.int32)

            def edge(e, accs):
                m = plsc.load_gather(cont_v, [jdx, jnp.full((16,), e,
                                                            jnp.int32)])
                new = tuple(
                    accs[q] * m + rows_v[e, pl.ds(16 * q, 16)]
                    for q in range(nv))
                for q in range(nv):
                    rows_v[e, pl.ds(16 * q, 16)] = new[q]
                return new

            accs = lax.fori_loop(0, _CHUNK, edge, accs)
            pltpu.sync_copy(rows_v, acc.at[scat_v.at[j]], add=True)
            return accs

        init = tuple(jnp.zeros((16,), jnp.float32) for _ in range(nv))
        lax.fori_loop(0, _K, window, init)
        plsc.subcore_barrier()

        for k in range(_WB // _CHUNK):
            pltpu.sync_copy(acc.at[pl.ds(base + k * _CHUNK, _CHUNK)], zbuf)
            pltpu.sync_copy(zbuf, out_hbm.at[c, pl.ds(base + k * _CHUNK,
                                                      _CHUNK)])
        rem = _WB - (_WB // _CHUNK) * _CHUNK
        pltpu.sync_copy(acc.at[pl.ds(base + _WB - rem, rem)],
                        zbuf.at[pl.ds(0, rem)])
        pltpu.sync_copy(zbuf.at[pl.ds(0, rem)],
                        out_hbm.at[c, pl.ds(base + _WB - rem, rem)])

    return scat(x, src_r, cont_r, scat_r)


_SIZES32 = [10368] + [9984] * 14 + [9856]
_SIZES128 = [10080] * 11 + [9840] * 4 + [9760]


def _edge_layout(src, dst, sizes):
    perm = jnp.argsort(dst, stable=True)
    ss = src[perm].astype(jnp.int32)
    dd = dst[perm].astype(jnp.int32)
    half = _E // 2
    srcs, conts, scats = [], [], []
    for h in range(2):
        bs = [h * half]
        for sz in sizes:
            bs.append(bs[-1] + sz)
        for t in range(16):
            lo, hi = bs[t], bs[t + 1]
            pad = _K * _CHUNK - (hi - lo)
            s_t, d_t = ss[lo:hi], dd[lo:hi]
            cont = jnp.concatenate(
                [jnp.zeros((1,), jnp.float32),
                 (d_t[1:] == d_t[:-1]).astype(jnp.float32)])
            runend = jnp.concatenate(
                [d_t[1:] != d_t[:-1], jnp.ones((1,), bool)])
            scat_i = jnp.where(runend, d_t, _N)
            srcs.append(jnp.pad(s_t, (0, pad)))
            conts.append(jnp.pad(cont, (0, pad)))
            scats.append(jnp.pad(scat_i, (0, pad), constant_values=_N))
    return (jnp.stack(srcs).reshape(_NTILES, _K, _CHUNK),
            jnp.stack(conts).reshape(_NTILES, _K, _CHUNK),
            jnp.stack(scats).reshape(_NTILES, _K, _CHUNK))


def _mid_body(x_ref, p_ref, wa_ref, ba_ref, wb_ref, bb_ref, o_ref):
    agg = p_ref[0][:_N] + p_ref[1][:_N]
    h0 = x_ref[...] + agg
    u = jnp.maximum(jnp.dot(h0, wa_ref[...],
                            preferred_element_type=jnp.float32)
                    + ba_ref[...], 0.0)
    o_ref[...] = jnp.dot(u, wb_ref[...],
                         preferred_element_type=jnp.float32) + bb_ref[...]


def _first_body(x_ref, pa_ref, pb_ref, wa_ref, ba_ref, wb_ref, bb_ref,
                o_ref):
    agg = jnp.concatenate(
        [pa_ref[0][:_N] + pa_ref[1][:_N], pb_ref[0][:_N] + pb_ref[1][:_N]],
        axis=1)
    h0 = x_ref[...] + agg
    u = jnp.maximum(jnp.dot(h0, wa_ref[...],
                            preferred_element_type=jnp.float32)
                    + ba_ref[...], 0.0)
    o_ref[...] = jnp.dot(u, wb_ref[...],
                         preferred_element_type=jnp.float32) + bb_ref[...]


def _norm_body(h_ref, m_ref, v_ref, g_ref, be_ref, o_ref):
    h = h_ref[...]
    o_ref[...] = jnp.maximum(
        (h - m_ref[...]) / jnp.sqrt(v_ref[...] + 1e-5) * g_ref[...]
        + be_ref[...], 0.0)


def _last_body(xn_ref, batch_ref, solv_ref, wg_ref, bg_ref, ws1_ref,
               bs1_ref, ws2_ref, bs2_ref, wf1a_ref, wf1b_ref, bf1_ref,
               wf2_ref, bf2_ref, wo_ref, bo_ref, o_ref):
    xn = xn_ref[...]
    ids = batch_ref[...]
    seg = lax.broadcasted_iota(jnp.int32, (_N, _G), 1)
    onehot = (ids == seg).astype(jnp.float32)
    pooled = lax.dot_general(onehot, xn, (((0,), (0,)), ((), ())),
                             preferred_element_type=jnp.float32,
                             precision=lax.Precision.HIGHEST)
    hg = jnp.maximum(jnp.dot(pooled, wg_ref[...],
                             preferred_element_type=jnp.float32)
                     + bg_ref[...], 0.0)
    s1 = jnp.maximum(jnp.dot(solv_ref[...], ws1_ref[...],
                             preferred_element_type=jnp.float32)
                     + bs1_ref[...], 0.0)
    s2 = jnp.maximum(jnp.dot(s1, ws2_ref[...],
                             preferred_element_type=jnp.float32)
                     + bs2_ref[...], 0.0)
    z1 = jnp.maximum(jnp.dot(hg, wf1a_ref[...],
                             preferred_element_type=jnp.float32)
                     + jnp.dot(s2, wf1b_ref[...],
                               preferred_element_type=jnp.float32)
                     + bf1_ref[...], 0.0)
    z2 = jnp.maximum(jnp.dot(z1, wf2_ref[...],
                             preferred_element_type=jnp.float32)
                     + bf2_ref[...], 0.0)
    o_ref[...] = (jnp.dot(z2, wo_ref[...], preferred_element_type=jnp.float32)
                  + bo_ref[...])


def kernel(x, edge_index, edge_attr, batch, solvent_fingerprint, params):
    p = params
    src1_r, cont1_r, scat1_r = _edge_layout(edge_index[0], edge_index[1],
                                            _SIZES128)
    src_r, cont_r, scat_r = _edge_layout(edge_index[0], edge_index[1],
                                         _SIZES32)
    batch2 = batch.reshape(_N, 1)

    def r1(a):
        return a.reshape(1, -1)

    def norm(h, i):
        m = jnp.mean(h, axis=0)
        v = jnp.var(h, axis=0)
        return pl.pallas_call(
            _norm_body,
            out_shape=jax.ShapeDtypeStruct((_N, _HID), jnp.float32),
        )(h, r1(m), r1(v), r1(p[f"g{i}"]), r1(p[f"be{i}"]))

    part_a = _sc_scatter_seq(x[:, :_HD], src1_r, cont1_r, scat1_r, _HD)
    part_b = _sc_scatter_seq(x[:, _HD:], src1_r, cont1_r, scat1_r, _HD)
    h = pl.pallas_call(
        _first_body,
        out_shape=jax.ShapeDtypeStruct((_N, _HID), jnp.float32),
    )(x, part_a, part_b, p["W1a"], r1(p["b1a"]), p["W1b"], r1(p["b1b"]))
    h = norm(h, 1)

    for i in range(2, 5):
        part = _sc_scatter_seq(h, src_r, cont_r, scat_r, _HID)
        h = pl.pallas_call(
            _mid_body,
            out_shape=jax.ShapeDtypeStruct((_N, _HID), jnp.float32),
        )(h, part, p[f"W{i}a"], r1(p[f"b{i}a"]), p[f"W{i}b"], r1(p[f"b{i}b"]))
        h = norm(h, i)

    part = _sc_scatter_seq(h, src_r, cont_r, scat_r, _HID)
    h = pl.pallas_call(
        _mid_body,
        out_shape=jax.ShapeDtypeStruct((_N, _HID), jnp.float32),
    )(h, part, p["W5a"], r1(p["b5a"]), p["W5b"], r1(p["b5b"]))
    h = norm(h, 5)
    out = pl.pallas_call(
        _last_body,
        out_shape=jax.ShapeDtypeStruct((_G, 1), jnp.float32),
    )(h, batch2, solvent_fingerprint,
      p["Wg"], r1(p["bg"]), p["Ws1"], r1(p["bs1"]), p["Ws2"], r1(p["bs2"]),
      p["Wf1"][:_G], p["Wf1"][_G:], r1(p["bf1"]), p["Wf2"], r1(p["bf2"]),
      p["Wo"], r1(p["bo"]))
    return out
